# baseline (speedup 1.0000x reference)
import functools

import jax
import jax.numpy as jnp
from jax import lax
from jax.experimental import pallas as pl
from jax.experimental.pallas import tpu as pltpu
from jax.experimental.pallas import tpu_sc as plsc

NUM_NODES = 1000000
MEMORY_DIM = 64
BATCH = 16384

_info = plsc.get_sparse_core_info()
_NC, _NS, _L = _info.num_cores, _info.num_subcores, _info.num_lanes
_NW = _NC * _NS
_BPW = BATCH // _NW


def _sc_gather_blend(idx, mem, ns, nf, ls, rate16, ct16):
    mesh = plsc.VectorSubcoreMesh(core_axis_name="c", subcore_axis_name="s")

    @functools.partial(
        pl.kernel,
        mesh=mesh,
        compiler_params=pltpu.CompilerParams(use_tc_tiling_on_sc=False),
        out_type=[
            jax.ShapeDtypeStruct((BATCH, MEMORY_DIM), jnp.float32),
            jax.ShapeDtypeStruct((BATCH, MEMORY_DIM), jnp.float32),
        ],
        scratch_types=[
            pltpu.VMEM((_BPW,), jnp.int32),
            pltpu.VMEM((_BPW, MEMORY_DIM), jnp.float32),
            pltpu.VMEM((_BPW, MEMORY_DIM), jnp.float32),
            pltpu.VMEM((_BPW, MEMORY_DIM), jnp.float32),
            pltpu.VMEM((_BPW,), jnp.float32),
            pltpu.VMEM((_BPW,), jnp.float32),
            pltpu.VMEM((_L,), jnp.float32),
            pltpu.VMEM((_L,), jnp.float32),
            pltpu.SemaphoreType.DMA,
        ],
    )
    def k(idx_hbm, mem_hbm, ns_hbm, nf_hbm, ls_hbm, rate_hbm, ct_hbm,
          a_out, f_out,
          idx_v, m_v, s_v, f_v, l_v, d_v, rate_v, ct_v, sem):
        wid = lax.axis_index("s") * _NC + lax.axis_index("c")
        base = wid * _BPW
        pltpu.sync_copy(idx_hbm.at[pl.ds(base, _BPW)], idx_v)
        cps = [
            pltpu.async_copy(mem_hbm.at[idx_v], m_v, sem),
            pltpu.async_copy(ns_hbm.at[idx_v], s_v, sem),
            pltpu.async_copy(nf_hbm.at[idx_v], f_v, sem),
            pltpu.async_copy(ls_hbm.at[idx_v], l_v, sem),
        ]
        pltpu.sync_copy(rate_hbm, rate_v)
        pltpu.sync_copy(ct_hbm, ct_v)
        for cp in cps:
            cp.wait()
        rate = rate_v[...]
        ct = ct_v[...]

        def dec_chunk(i, _):
            sl = pl.ds(i * _L, _L)
            d_v[sl] = jnp.exp(rate * (ct - l_v[sl]))
            return 0

        lax.fori_loop(0, _BPW // _L, dec_chunk, 0, unroll=4)

        def blend_chunk(c, _):
            dvec = d_v[pl.ds(c * _L, _L)]
            for j in range(_L):
                r = c * _L + j
                dv = jnp.full((_L,), dvec[j], jnp.float32)
                for q in range(MEMORY_DIM // _L):
                    sl = pl.ds(q * _L, _L)
                    m_v[r, sl] = m_v[r, sl] + s_v[r, sl] * dv
            return 0

        lax.fori_loop(0, _BPW // _L, blend_chunk, 0)
        pltpu.sync_copy(m_v, a_out.at[pl.ds(base, _BPW)])
        pltpu.sync_copy(f_v, f_out.at[pl.ds(base, _BPW)])

    return k(idx, mem, ns, nf, ls, rate16, ct16)


def _tc_body(a_ref, f_ref, nfw_ref, nfb_ref, c1w_ref, c1b_ref, c2w_ref,
             c2b_ref, o_ref):
    blended = a_ref[...] + jnp.dot(
        f_ref[...], nfw_ref[...], preferred_element_type=jnp.float32
    ) + nfb_ref[...]
    h = jnp.maximum(
        jnp.dot(blended, c1w_ref[...], preferred_element_type=jnp.float32)
        + c1b_ref[...], 0.0)
    o_ref[...] = jnp.dot(
        h, c2w_ref[...], preferred_element_type=jnp.float32) + c2b_ref[...]


def kernel(label_nodes, current_time, memory_state, node_state, last_seen,
           node_features, decay, nf_W, nf_b, c1_W, c1_b, c2_W, c2_b):
    rate = -jax.nn.relu(decay)
    ct = jnp.asarray(current_time, jnp.float32)
    rate16 = jnp.full((_L,), rate, jnp.float32)
    ct16 = jnp.full((_L,), ct, jnp.float32)
    idx = label_nodes.astype(jnp.int32)

    a, f = _sc_gather_blend(idx, memory_state, node_state, node_features,
                            last_seen, rate16, ct16)

    logits = pl.pallas_call(
        _tc_body,
        out_shape=jax.ShapeDtypeStruct((BATCH, MEMORY_DIM), jnp.float32),
    )(a, f, nf_W, nf_b.reshape(1, MEMORY_DIM), c1_W,
      c1_b.reshape(1, MEMORY_DIM), c2_W, c2_b.reshape(1, MEMORY_DIM))
    return logits


# speedup vs baseline: 1.5146x; 1.5146x over previous
import functools

import jax
import jax.numpy as jnp
from jax import lax
from jax.experimental import pallas as pl
from jax.experimental.pallas import tpu as pltpu
from jax.experimental.pallas import tpu_sc as plsc

NUM_NODES = 1000000
MEMORY_DIM = 64
BATCH = 16384

_info = plsc.get_sparse_core_info()
_NC, _NS, _L = _info.num_cores, _info.num_subcores, _info.num_lanes
_NW = _NC * _NS
_BPW = BATCH // _NW
_BPC = _BPW // 2


def _sc_dec(idx, ls, rate16, ct16):
    mesh = plsc.VectorSubcoreMesh(core_axis_name="c", subcore_axis_name="s")

    @functools.partial(
        pl.kernel,
        mesh=mesh,
        compiler_params=pltpu.CompilerParams(use_tc_tiling_on_sc=False),
        out_type=jax.ShapeDtypeStruct((BATCH,), jnp.float32),
        scratch_types=[
            pltpu.VMEM((_BPW,), jnp.int32),
            pltpu.VMEM((_BPW,), jnp.float32),
            pltpu.VMEM((_BPW,), jnp.float32),
            pltpu.VMEM((_L,), jnp.float32),
            pltpu.VMEM((_L,), jnp.float32),
            pltpu.SemaphoreType.DMA,
        ],
    )
    def k(idx_hbm, ls_hbm, rate_hbm, ct_hbm, dec_out,
          idx_v, l_v, d_v, rate_v, ct_v, sem):
        wid = lax.axis_index("s") * _NC + lax.axis_index("c")
        base = wid * _BPW
        pltpu.sync_copy(idx_hbm.at[pl.ds(base, _BPW)], idx_v)
        cp = pltpu.async_copy(ls_hbm.at[idx_v], l_v, sem)
        pltpu.sync_copy(rate_hbm, rate_v)
        pltpu.sync_copy(ct_hbm, ct_v)
        cp.wait()
        rate = rate_v[...]
        ct = ct_v[...]

        def dec_chunk(i, _):
            sl = pl.ds(i * _L, _L)
            d_v[sl] = jnp.exp(rate * (ct - l_v[sl]))
            return 0

        lax.fori_loop(0, _BPW // _L, dec_chunk, 0, unroll=4)
        pltpu.sync_copy(d_v, dec_out.at[pl.ds(base, _BPW)])

    return k(idx, ls, rate16, ct16)


def _sc_gather_blend(idx, dec, mem, ns, nf):
    mesh = plsc.VectorSubcoreMesh(core_axis_name="c", subcore_axis_name="s")

    @functools.partial(
        pl.kernel,
        mesh=mesh,
        out_type=[
            jax.ShapeDtypeStruct((BATCH, MEMORY_DIM), jnp.float32),
            jax.ShapeDtypeStruct((BATCH, MEMORY_DIM), jnp.float32),
        ],
        scratch_types=[
            pltpu.VMEM((_BPW,), jnp.int32),
            pltpu.VMEM((_BPW,), jnp.float32),
            pltpu.VMEM((_BPC, MEMORY_DIM), jnp.float32),
            pltpu.VMEM((_BPC, MEMORY_DIM), jnp.float32),
            pltpu.VMEM((_BPC, MEMORY_DIM), jnp.float32),
            pltpu.SemaphoreType.DMA,
        ],
    )
    def k(idx_hbm, dec_hbm, mem_hbm, ns_hbm, nf_hbm,
          a_out, f_out,
          idx_v, d_v, m_v, s_v, f_v, sem):
        wid = lax.axis_index("s") * _NC + lax.axis_index("c")
        base = wid * _BPW
        pltpu.sync_copy(idx_hbm.at[pl.ds(base, _BPW)], idx_v)
        pltpu.sync_copy(dec_hbm.at[pl.ds(base, _BPW)], d_v)

        for chunk in range(_BPW // _BPC):
            coff = chunk * _BPC

            def fire(c, _):
                ivec = idx_v[pl.ds(coff + c * _L, _L)]
                for j in range(_L):
                    n = ivec[j]
                    lr = c * _L + j
                    pltpu.async_copy(mem_hbm.at[n], m_v.at[lr], sem)
                    pltpu.async_copy(ns_hbm.at[n], s_v.at[lr], sem)
                    pltpu.async_copy(nf_hbm.at[n], f_v.at[lr], sem)
                return 0

            lax.fori_loop(0, _BPC // _L, fire, 0)
            pltpu.make_async_copy(mem_hbm.at[pl.ds(0, _BPC)], m_v, sem).wait()
            pltpu.make_async_copy(ns_hbm.at[pl.ds(0, _BPC)], s_v, sem).wait()
            pltpu.make_async_copy(nf_hbm.at[pl.ds(0, _BPC)], f_v, sem).wait()

            def blend_chunk(c, _):
                dvec = d_v[pl.ds(coff + c * _L, _L)]
                for j in range(_L):
                    lr = c * _L + j
                    dv = jnp.full((_L,), dvec[j], jnp.float32)
                    for q in range(MEMORY_DIM // _L):
                        sl = pl.ds(q * _L, _L)
                        m_v[lr, sl] = m_v[lr, sl] + s_v[lr, sl] * dv
                return 0

            lax.fori_loop(0, _BPC // _L, blend_chunk, 0)
            pltpu.sync_copy(m_v, a_out.at[pl.ds(base + coff, _BPC)])
            pltpu.sync_copy(f_v, f_out.at[pl.ds(base + coff, _BPC)])

    return k(idx, dec, mem, ns, nf)


def _tc_body(a_ref, f_ref, nfw_ref, nfb_ref, c1w_ref, c1b_ref, c2w_ref,
             c2b_ref, o_ref):
    blended = a_ref[...] + jnp.dot(
        f_ref[...], nfw_ref[...], preferred_element_type=jnp.float32
    ) + nfb_ref[...]
    h = jnp.maximum(
        jnp.dot(blended, c1w_ref[...], preferred_element_type=jnp.float32)
        + c1b_ref[...], 0.0)
    o_ref[...] = jnp.dot(
        h, c2w_ref[...], preferred_element_type=jnp.float32) + c2b_ref[...]


def kernel(label_nodes, current_time, memory_state, node_state, last_seen,
           node_features, decay, nf_W, nf_b, c1_W, c1_b, c2_W, c2_b):
    rate = -jax.nn.relu(decay)
    ct = jnp.asarray(current_time, jnp.float32)
    rate16 = jnp.full((_L,), rate, jnp.float32)
    ct16 = jnp.full((_L,), ct, jnp.float32)
    idx = label_nodes.astype(jnp.int32)

    dec = _sc_dec(idx, last_seen, rate16, ct16)
    a, f = _sc_gather_blend(idx, dec, memory_state, node_state, node_features)

    logits = pl.pallas_call(
        _tc_body,
        out_shape=jax.ShapeDtypeStruct((BATCH, MEMORY_DIM), jnp.float32),
    )(a, f, nf_W, nf_b.reshape(1, MEMORY_DIM), c1_W,
      c1_b.reshape(1, MEMORY_DIM), c2_W, c2_b.reshape(1, MEMORY_DIM))
    return logits


# speedup vs baseline: 3.4065x; 2.2491x over previous
import functools

import jax
import jax.numpy as jnp
from jax import lax
from jax.experimental import pallas as pl
from jax.experimental.pallas import tpu as pltpu
from jax.experimental.pallas import tpu_sc as plsc

NUM_NODES = 1000000
MEMORY_DIM = 64
BATCH = 16384

_NB = 8192
_NPAD = 1007616
_GRID = _NPAD // _NB

_info = plsc.get_sparse_core_info()
_NC, _NS, _L = _info.num_cores, _info.num_subcores, _info.num_lanes
_NW = _NC * _NS
_BPW = BATCH // _NW


def _mlp_body(mem_ref, ns_ref, nf_ref, ls_ref, rc_ref,
              nfw_ref, nfb_ref, c1w_ref, c1b_ref, c2w_ref, c2b_ref, o_ref):
    rate = rc_ref[0, 0]
    ct = rc_ref[0, 1]
    dec = jnp.exp(rate * (ct - ls_ref[...]))
    blended = mem_ref[...] + ns_ref[...] * dec + jnp.dot(
        nfw_ref[...], nf_ref[...], preferred_element_type=jnp.float32
    ) + nfb_ref[...]
    h = jnp.maximum(
        jnp.dot(c1w_ref[...], blended, preferred_element_type=jnp.float32)
        + c1b_ref[...], 0.0)
    logits_t = jnp.dot(
        c2w_ref[...], h, preferred_element_type=jnp.float32) + c2b_ref[...]
    o_ref[...] = logits_t.T


def _tc_full_mlp(mem_t, ns_t, nf_t, ls2, rc, nf_Wt, nf_b, c1_Wt, c1_b,
                 c2_Wt, c2_b):
    wspec = pl.BlockSpec((MEMORY_DIM, MEMORY_DIM), lambda i: (0, 0))
    bspec = pl.BlockSpec((MEMORY_DIM, 1), lambda i: (0, 0))
    tspec = pl.BlockSpec((MEMORY_DIM, _NB), lambda i: (0, i))
    return pl.pallas_call(
        _mlp_body,
        grid=(_GRID,),
        in_specs=[
            tspec, tspec, tspec,
            pl.BlockSpec((1, _NB), lambda i: (0, i)),
            pl.BlockSpec((1, 2), lambda i: (0, 0)),
            wspec, bspec, wspec, bspec, wspec, bspec,
        ],
        out_specs=pl.BlockSpec((_NB, MEMORY_DIM), lambda i: (i, 0)),
        out_shape=jax.ShapeDtypeStruct((_NPAD, MEMORY_DIM), jnp.float32),
    )(mem_t, ns_t, nf_t, ls2, rc, nf_Wt, nf_b, c1_Wt, c1_b, c2_Wt, c2_b)


def _sc_gather(idx, table):
    mesh = plsc.VectorSubcoreMesh(core_axis_name="c", subcore_axis_name="s")

    @functools.partial(
        pl.kernel,
        mesh=mesh,
        out_type=jax.ShapeDtypeStruct((BATCH, MEMORY_DIM), jnp.float32),
        scratch_types=[
            pltpu.VMEM((_BPW,), jnp.int32),
            pltpu.VMEM((_BPW, MEMORY_DIM), jnp.float32),
            pltpu.SemaphoreType.DMA,
        ],
    )
    def k(idx_hbm, tab_hbm, out_hbm, idx_v, r_v, sem):
        wid = lax.axis_index("s") * _NC + lax.axis_index("c")
        base = wid * _BPW
        pltpu.sync_copy(idx_hbm.at[pl.ds(base, _BPW)], idx_v)

        def fire(c, _):
            ivec = idx_v[pl.ds(c * _L, _L)]
            for j in range(_L):
                pltpu.async_copy(
                    tab_hbm.at[ivec[j]], r_v.at[c * _L + j], sem)
            return 0

        lax.fori_loop(0, _BPW // _L, fire, 0)
        pltpu.make_async_copy(tab_hbm.at[pl.ds(0, _BPW)], r_v, sem).wait()
        pltpu.sync_copy(r_v, out_hbm.at[pl.ds(base, _BPW)])

    return k(idx, table)


def kernel(label_nodes, current_time, memory_state, node_state, last_seen,
           node_features, decay, nf_W, nf_b, c1_W, c1_b, c2_W, c2_b):
    rate = -jax.nn.relu(decay)
    ct = jnp.asarray(current_time, jnp.float32)
    rc = jnp.stack([rate, ct]).reshape(1, 2)
    idx = label_nodes.astype(jnp.int32)

    logits_full = _tc_full_mlp(
        memory_state.T, node_state.T, node_features.T,
        last_seen.reshape(1, NUM_NODES), rc,
        nf_W.T, nf_b.reshape(MEMORY_DIM, 1),
        c1_W.T, c1_b.reshape(MEMORY_DIM, 1),
        c2_W.T, c2_b.reshape(MEMORY_DIM, 1))

    return _sc_gather(idx, logits_full)
